# baseline (speedup 1.0000x reference)
import functools

import jax
import jax.numpy as jnp
from jax import lax
from jax.experimental import pallas as pl
from jax.experimental.pallas import tpu as pltpu
from jax.experimental.pallas import tpu_sc as plsc

N_ROWS = 16384
N_COLS = 4096
NUM_CORES = 2
NUM_SUBCORES = 16
NW = NUM_CORES * NUM_SUBCORES
ROWS_PER_W = N_ROWS // NW
LANES = 16
PERIOD = 128
CHUNK_ROWS = 4
N_CHUNKS = ROWS_PER_W // CHUNK_ROWS
N_GROUPS = N_CHUNKS // 2


_NEG_LOG2E = -1.4426950408889634


def _apply_acts(src, dst):
    for r in range(CHUNK_ROWS):

        @plsc.parallel_loop(0, N_COLS, step=PERIOD, unroll=2)
        def body(base, r=r):
            for v in range(4):
                s = base + v * LANES
                x = src[r, pl.ds(s, LANES)]
                dst[r, pl.ds(s, LANES)] = jnp.maximum(x, 0.0)
            for v in range(4):
                s = base + 64 + v * LANES
                x = src[r, pl.ds(s, LANES)]
                dst[r, pl.ds(s, LANES)] = x / (1.0 + jnp.exp(x * -1.0))


_MESH = plsc.VectorSubcoreMesh(core_axis_name="c", subcore_axis_name="s")


@functools.partial(
    pl.kernel,
    mesh=_MESH,
    out_type=jax.ShapeDtypeStruct((N_ROWS, N_COLS), jnp.float32),
    scratch_types=[
        pltpu.VMEM((CHUNK_ROWS, N_COLS), jnp.float32),
        pltpu.VMEM((CHUNK_ROWS, N_COLS), jnp.float32),
        pltpu.VMEM((CHUNK_ROWS, N_COLS), jnp.float32),
        pltpu.VMEM((CHUNK_ROWS, N_COLS), jnp.float32),
        pltpu.SemaphoreType.DMA,
        pltpu.SemaphoreType.DMA,
        pltpu.SemaphoreType.DMA,
        pltpu.SemaphoreType.DMA,
    ],
)
def _mixed_act_sc(x_hbm, out_hbm, ib0, ib1, ob0, ob1, is0, is1, os0, os1):
    wid = lax.axis_index("s") * NUM_CORES + lax.axis_index("c")
    base_row = wid * ROWS_PER_W

    def _src(i):
        return x_hbm.at[pl.ds(base_row + i * CHUNK_ROWS, CHUNK_ROWS), :]

    def _dst(i):
        return out_hbm.at[pl.ds(base_row + i * CHUNK_ROWS, CHUNK_ROWS), :]

    pltpu.async_copy(_src(0), ib0, is0)
    pltpu.async_copy(_src(1), ib1, is1)

    def group(g, carry):
        for b, (ib, ob, isem, osem) in enumerate(
            ((ib0, ob0, is0, os0), (ib1, ob1, is1, os1))
        ):
            i = 2 * g + b
            pltpu.make_async_copy(_src(i), ib, isem).wait()
            @pl.when(g > 0)
            def _wait_store():
                pltpu.make_async_copy(ob, _dst(i), osem).wait()

            _apply_acts(ib, ob)
            pltpu.async_copy(ob, _dst(i), osem)

            @pl.when(g < N_GROUPS - 1)
            def _next_load():
                pltpu.async_copy(_src(i + 2), ib, isem)

        return carry

    lax.fori_loop(0, N_GROUPS, group, 0)

    pltpu.make_async_copy(ob0, _dst(N_CHUNKS - 2), os0).wait()
    pltpu.make_async_copy(ob1, _dst(N_CHUNKS - 1), os1).wait()


def kernel(input_tensor):
    return _mixed_act_sc(input_tensor)


# speedup vs baseline: 1.2853x; 1.2853x over previous
import functools

import jax
import jax.numpy as jnp
from jax import lax
from jax.experimental import pallas as pl
from jax.experimental.pallas import tpu as pltpu
from jax.experimental.pallas import tpu_sc as plsc

N_ROWS = 16384
N_COLS = 4096
NUM_CORES = 2
NUM_SUBCORES = 16
NW = NUM_CORES * NUM_SUBCORES
ROWS_PER_W = N_ROWS // NW
LANES = 16
PERIOD = 128
CHUNK_ROWS = 4
N_CHUNKS = ROWS_PER_W // CHUNK_ROWS
N_GROUPS = N_CHUNKS // 2


_NEG_LOG2E = -1.4426950408889634


def _apply_acts(src, dst):
    for r in range(CHUNK_ROWS):

        @plsc.parallel_loop(0, N_COLS, step=PERIOD)
        def body(base, r=r):
            for v in range(4):
                s = base + v * LANES
                x = src[r, pl.ds(s, LANES)]
                dst[r, pl.ds(s, LANES)] = jnp.maximum(x, 0.0)
            for v in range(4):
                s = base + 64 + v * LANES
                x = src[r, pl.ds(s, LANES)]
                dst[r, pl.ds(s, LANES)] = x / (1.0 + jnp.exp(x * -1.0))


_MESH = plsc.VectorSubcoreMesh(core_axis_name="c", subcore_axis_name="s")


@functools.partial(
    pl.kernel,
    mesh=_MESH,
    out_type=jax.ShapeDtypeStruct((N_ROWS, N_COLS), jnp.float32),
    scratch_types=[
        pltpu.VMEM((CHUNK_ROWS, N_COLS), jnp.float32),
        pltpu.VMEM((CHUNK_ROWS, N_COLS), jnp.float32),
        pltpu.VMEM((CHUNK_ROWS, N_COLS), jnp.float32),
        pltpu.VMEM((CHUNK_ROWS, N_COLS), jnp.float32),
        pltpu.SemaphoreType.DMA,
        pltpu.SemaphoreType.DMA,
        pltpu.SemaphoreType.DMA,
        pltpu.SemaphoreType.DMA,
    ],
)
def _mixed_act_sc(x_hbm, out_hbm, ib0, ib1, ob0, ob1, is0, is1, os0, os1):
    wid = lax.axis_index("s") * NUM_CORES + lax.axis_index("c")
    base_row = wid * ROWS_PER_W

    def _src(i):
        return x_hbm.at[pl.ds(base_row + i * CHUNK_ROWS, CHUNK_ROWS), :]

    def _dst(i):
        return out_hbm.at[pl.ds(base_row + i * CHUNK_ROWS, CHUNK_ROWS), :]

    pltpu.async_copy(_src(0), ib0, is0)
    pltpu.async_copy(_src(1), ib1, is1)

    def group(g, carry):
        for b, (ib, ob, isem, osem) in enumerate(
            ((ib0, ob0, is0, os0), (ib1, ob1, is1, os1))
        ):
            i = 2 * g + b
            pltpu.make_async_copy(_src(i), ib, isem).wait()
            @pl.when(g > 0)
            def _wait_store():
                pltpu.make_async_copy(ob, _dst(i), osem).wait()

            _apply_acts(ib, ob)
            pltpu.async_copy(ob, _dst(i), osem)

            @pl.when(g < N_GROUPS - 1)
            def _next_load():
                pltpu.async_copy(_src(i + 2), ib, isem)

        return carry

    lax.fori_loop(0, N_GROUPS, group, 0)

    pltpu.make_async_copy(ob0, _dst(N_CHUNKS - 2), os0).wait()
    pltpu.make_async_copy(ob1, _dst(N_CHUNKS - 1), os1).wait()


def kernel(input_tensor):
    return _mixed_act_sc(input_tensor)
